# baseline (speedup 1.0000x reference)
import functools

import jax
import jax.numpy as jnp
from jax import lax
from jax.experimental import pallas as pl
from jax.experimental.pallas import tpu as pltpu
from jax.experimental.pallas import tpu_sc as plsc

B_Q = 1024
D = 128
N_KEYS = 100000
BLK = 2048
N_BLK = 49
M_PAD = N_BLK * BLK
CHUNK = 128
N_CHUNK = M_PAD // CHUNK
N_CHUNK_PAD = 896
TOPK = 10
NEG = -3.0e38
BIG_I = 2**30

SC_CORES = 2
SC_SUBCORES = 16
NW = SC_CORES * SC_SUBCORES
N_IDX = B_Q * TOPK
IDX_PER_W = N_IDX // NW
IDX_ROWS = 3
IDX_PAD_W = IDX_ROWS * CHUNK


def _simblock_kernel(q_ref, k_ref, sim_ref, cmax_ref, qn_ref):
    j = pl.program_id(0)

    @pl.when(j == 0)
    def _():
        q = q_ref[...]
        n = jnp.sqrt(jnp.sum(q * q, axis=1, keepdims=True))
        qn_ref[...] = q / jnp.maximum(n, 1e-12)

    k = k_ref[...]
    kn = k / jnp.maximum(jnp.sqrt(jnp.sum(k * k, axis=1, keepdims=True)), 1e-12)
    qn = qn_ref[...]
    parts = []
    s_chunks = []
    for c in range(BLK // CHUNK):
        s_c = lax.dot_general(qn, kn[c * CHUNK:(c + 1) * CHUNK, :],
                              (((1,), (1,)), ((), ())),
                              preferred_element_type=jnp.float32)
        s_chunks.append(s_c)
        sim_ref[0, c] = s_c
        parts.append(jnp.max(s_c, axis=1, keepdims=True))
    cmax_ref[0] = jnp.concatenate(parts, axis=1)

    @pl.when(j == N_BLK - 1)
    def _():
        lane = lax.broadcasted_iota(jnp.int32, (B_Q, CHUNK), 1)
        mparts = list(parts)
        c_lo = (N_KEYS - (N_BLK - 1) * BLK) // CHUNK
        for c in range(c_lo, BLK // CHUNK):
            col = (N_BLK - 1) * BLK + c * CHUNK + lane
            s_m = jnp.where(col < N_KEYS, s_chunks[c], NEG)
            sim_ref[0, c] = s_m
            mparts[c] = jnp.max(s_m, axis=1, keepdims=True)
        cmax_ref[0] = jnp.concatenate(mparts, axis=1)


def _topchunk_kernel(cmax_ref, cidx_ref):
    parts = [cmax_ref[c] for c in range(N_BLK)]
    parts.append(jnp.full((B_Q, N_CHUNK_PAD - N_CHUNK), NEG, jnp.float32))
    v = jnp.concatenate(parts, axis=1)
    col = lax.broadcasted_iota(jnp.int32, (B_Q, N_CHUNK_PAD), 1)
    ocol = lax.broadcasted_iota(jnp.int32, (B_Q, CHUNK), 1)
    out = jnp.zeros((B_Q, CHUNK), jnp.int32)
    for t in range(TOPK):
        m = jnp.max(v, axis=1, keepdims=True)
        sel = jnp.where(v == m, col, BIG_I)
        idx = jnp.min(sel, axis=1, keepdims=True)
        out = jnp.where(ocol == t, idx, out)
        v = jnp.where(col == idx, NEG, v)
    cidx_ref[...] = out


def _final_topk_kernel(cand_ref, cidx_ref, vout_ref, iout_ref):
    cand = cand_ref[...]
    lane = lax.broadcasted_iota(jnp.int32, (B_Q, CHUNK), 1)
    keyidx = jnp.concatenate(
        [cidx_ref[:, t:t + 1] * CHUNK + lane for t in range(TOPK)], axis=1)
    ocol = lax.broadcasted_iota(jnp.int32, (B_Q, CHUNK), 1)
    vout = jnp.zeros((B_Q, CHUNK), jnp.float32)
    iout = jnp.zeros((B_Q, CHUNK), jnp.int32)
    for t in range(TOPK):
        m = jnp.max(cand, axis=1, keepdims=True)
        sel = jnp.where(cand == m, keyidx, BIG_I)
        ki = jnp.min(sel, axis=1, keepdims=True)
        vout = jnp.where(ocol == t, m, vout)
        iout = jnp.where(ocol == t, ki, iout)
        cand = jnp.where((cand == m) & (keyidx == ki), NEG, cand)
    vout_ref[...] = vout
    iout_ref[...] = iout


def _sc_gather(table, idx3d):
    d = table.shape[1]
    mesh = plsc.VectorSubcoreMesh(core_axis_name="c", subcore_axis_name="s")

    @functools.partial(
        pl.kernel, mesh=mesh,
        out_type=jax.ShapeDtypeStruct((NW * IDX_PAD_W, d), jnp.float32),
        scratch_types=[
            pltpu.VMEM((IDX_ROWS, CHUNK), jnp.int32),
            pltpu.VMEM((IDX_PAD_W, d), jnp.float32),
            pltpu.SemaphoreType.DMA,
        ],
    )
    def gather(table_hbm, idx_hbm, out_hbm, idx_v, rows_v, sem):
        wid = lax.axis_index("s") * SC_CORES + lax.axis_index("c")
        pltpu.sync_copy(idx_hbm.at[wid], idx_v)
        copies = [
            pltpu.async_copy(table_hbm.at[idx_v.at[j]],
                             rows_v.at[pl.ds(j * CHUNK, CHUNK)], sem)
            for j in range(IDX_ROWS)
        ]
        for cp in copies:
            cp.wait()
        pltpu.sync_copy(rows_v, out_hbm.at[pl.ds(wid * IDX_PAD_W, IDX_PAD_W)])

    return gather(table, idx3d)


def _pad_worker_idx(flat_idx):
    per_w = flat_idx.reshape(NW, IDX_PER_W)
    return jnp.pad(per_w, ((0, 0), (0, IDX_PAD_W - IDX_PER_W))).reshape(
        NW, IDX_ROWS, CHUNK)


def _unpad_worker_rows(rows, d):
    return rows.reshape(NW, IDX_PAD_W, d)[:, :IDX_PER_W].reshape(N_IDX, d)


def kernel(query_embeddings, key_embeddings):
    q = query_embeddings.reshape(B_Q, D)
    k = key_embeddings.reshape(N_KEYS, D)

    sim, cmax = pl.pallas_call(
        _simblock_kernel,
        grid=(N_BLK,),
        in_specs=[
            pl.BlockSpec((B_Q, D), lambda j: (0, 0)),
            pl.BlockSpec((BLK, D), lambda j: (j, 0)),
        ],
        out_specs=[
            pl.BlockSpec((1, BLK // CHUNK, B_Q, CHUNK), lambda j: (j, 0, 0, 0)),
            pl.BlockSpec((1, B_Q, BLK // CHUNK), lambda j: (j, 0, 0)),
        ],
        out_shape=[
            jax.ShapeDtypeStruct((N_BLK, BLK // CHUNK, B_Q, CHUNK), jnp.float32),
            jax.ShapeDtypeStruct((N_BLK, B_Q, BLK // CHUNK), jnp.float32),
        ],
        scratch_shapes=[pltpu.VMEM((B_Q, D), jnp.float32)],
    )(q, k)

    cidx_pad = pl.pallas_call(
        _topchunk_kernel,
        out_shape=jax.ShapeDtypeStruct((B_Q, CHUNK), jnp.int32),
    )(cmax)

    chunk_idx = cidx_pad[:, :TOPK]
    row = jnp.arange(B_Q, dtype=jnp.int32)[:, None]
    flat_cand = (chunk_idx * B_Q + row).reshape(-1)

    sim_rows = sim.reshape(B_Q * N_CHUNK, CHUNK)
    cand = _unpad_worker_rows(
        _sc_gather(sim_rows, _pad_worker_idx(flat_cand)), CHUNK)

    vout, iout = pl.pallas_call(
        _final_topk_kernel,
        out_shape=[
            jax.ShapeDtypeStruct((B_Q, CHUNK), jnp.float32),
            jax.ShapeDtypeStruct((B_Q, CHUNK), jnp.int32),
        ],
    )(cand.reshape(B_Q, TOPK * CHUNK), cidx_pad)

    top_sim = vout[:, :TOPK]
    top_idx = iout[:, :TOPK]

    retrieved = _unpad_worker_rows(
        _sc_gather(k, _pad_worker_idx(top_idx.reshape(-1))), D)
    return top_sim, top_idx, retrieved.reshape(B_Q, TOPK, 1, D)


# speedup vs baseline: 1.5733x; 1.5733x over previous
import functools

import jax
import jax.numpy as jnp
from jax import lax
from jax.experimental import pallas as pl
from jax.experimental.pallas import tpu as pltpu
from jax.experimental.pallas import tpu_sc as plsc

B_Q = 1024
D = 128
N_KEYS = 100000
BLK = 2048
N_BLK = 49
M_PAD = N_BLK * BLK
CHUNK = 128
N_CHUNK = M_PAD // CHUNK
N_CHUNK_PAD = 896
TOPK = 10
NEG = -3.0e38
BIG_I = 2**30

SC_CORES = 2
SC_SUBCORES = 16
NW = SC_CORES * SC_SUBCORES
N_IDX = B_Q * TOPK
IDX_PER_W = N_IDX // NW
IDX_ROWS = 6
IDX_PAD_W = IDX_ROWS * CHUNK


def _simblock_kernel(q_ref, k_ref, sim_ref, cmax_ref, qn_ref):
    j = pl.program_id(0)

    @pl.when(j == 0)
    def _():
        q = q_ref[...]
        n = jnp.sqrt(jnp.sum(q * q, axis=1, keepdims=True))
        qn_ref[...] = q / jnp.maximum(n, 1e-12)

    k = k_ref[...]
    kn = k / jnp.maximum(jnp.sqrt(jnp.sum(k * k, axis=1, keepdims=True)), 1e-12)
    qn = qn_ref[...]
    parts = []
    s_chunks = []
    for c in range(BLK // CHUNK):
        s_c = lax.dot_general(qn, kn[c * CHUNK:(c + 1) * CHUNK, :],
                              (((1,), (1,)), ((), ())),
                              preferred_element_type=jnp.float32)
        s_chunks.append(s_c)
        sim_ref[0, c] = s_c
        parts.append(jnp.max(s_c, axis=1, keepdims=True))
    cmax_ref[0] = jnp.concatenate(parts, axis=1)

    @pl.when(j == N_BLK - 1)
    def _():
        lane = lax.broadcasted_iota(jnp.int32, (B_Q, CHUNK), 1)
        mparts = list(parts)
        c_lo = (N_KEYS - (N_BLK - 1) * BLK) // CHUNK
        for c in range(c_lo, BLK // CHUNK):
            col = (N_BLK - 1) * BLK + c * CHUNK + lane
            s_m = jnp.where(col < N_KEYS, s_chunks[c], NEG)
            sim_ref[0, c] = s_m
            mparts[c] = jnp.max(s_m, axis=1, keepdims=True)
        cmax_ref[0] = jnp.concatenate(mparts, axis=1)


def _topchunk_kernel(cmax_ref, cidx_ref):
    parts = [cmax_ref[c] for c in range(N_BLK)]
    parts.append(jnp.full((B_Q, N_CHUNK_PAD - N_CHUNK), NEG, jnp.float32))
    v = jnp.concatenate(parts, axis=1)
    col = lax.broadcasted_iota(jnp.int32, (B_Q, N_CHUNK_PAD), 1)
    ocol = lax.broadcasted_iota(jnp.int32, (B_Q, CHUNK), 1)
    out = jnp.zeros((B_Q, CHUNK), jnp.int32)
    for t in range(TOPK):
        m = jnp.max(v, axis=1, keepdims=True)
        sel = jnp.where(v == m, col, BIG_I)
        idx = jnp.min(sel, axis=1, keepdims=True)
        out = jnp.where(ocol == t, idx, out)
        v = jnp.where(col == idx, NEG, v)
    cidx_ref[...] = out


def _final_topk_kernel(cand_ref, cidx_ref, vout_ref, iout_ref):
    cand = cand_ref[...]
    lane = lax.broadcasted_iota(jnp.int32, (B_Q, CHUNK), 1)
    keyidx = jnp.concatenate(
        [cidx_ref[:, t:t + 1] * CHUNK + lane for t in range(TOPK)], axis=1)
    ocol = lax.broadcasted_iota(jnp.int32, (B_Q, CHUNK), 1)
    vout = jnp.zeros((B_Q, CHUNK), jnp.float32)
    iout = jnp.zeros((B_Q, CHUNK), jnp.int32)
    for t in range(TOPK):
        m = jnp.max(cand, axis=1, keepdims=True)
        sel = jnp.where(cand == m, keyidx, BIG_I)
        ki = jnp.min(sel, axis=1, keepdims=True)
        vout = jnp.where(ocol == t, m, vout)
        iout = jnp.where(ocol == t, ki, iout)
        cand = jnp.where((cand == m) & (keyidx == ki), NEG, cand)
    vout_ref[...] = vout
    iout_ref[...] = iout


def _sc_gather(table, idx3d):
    d = table.shape[1]
    mesh = plsc.VectorSubcoreMesh(core_axis_name="c", subcore_axis_name="s")

    @functools.partial(
        pl.kernel, mesh=mesh,
        out_type=jax.ShapeDtypeStruct((NW * IDX_PAD_W, d), jnp.float32),
        scratch_types=[
            pltpu.VMEM((IDX_ROWS, CHUNK), jnp.int32),
            pltpu.VMEM((IDX_PAD_W, d), jnp.float32),
            pltpu.SemaphoreType.DMA,
        ],
    )
    def gather(table_hbm, idx_hbm, out_hbm, idx_v, rows_v, sem):
        wid = lax.axis_index("s") * SC_CORES + lax.axis_index("c")
        pltpu.sync_copy(idx_hbm.at[wid], idx_v)
        copies = [
            pltpu.async_copy(table_hbm.at[idx_v.at[j]],
                             rows_v.at[pl.ds(j * CHUNK, CHUNK)], sem)
            for j in range(IDX_ROWS)
        ]
        for cp in copies:
            cp.wait()
        pltpu.sync_copy(rows_v, out_hbm.at[pl.ds(wid * IDX_PAD_W, IDX_PAD_W)])

    return gather(table, idx3d)


def _pad_worker_idx(flat_idx):
    per_w = flat_idx.reshape(NW, IDX_PER_W)
    return jnp.pad(per_w, ((0, 0), (0, IDX_PAD_W - IDX_PER_W)),
                   mode="wrap").reshape(NW, IDX_ROWS, CHUNK)


def _unpad_worker_rows(rows, d):
    return rows.reshape(NW, IDX_PAD_W, d)[:, :IDX_PER_W].reshape(N_IDX, d)


def kernel(query_embeddings, key_embeddings):
    q = query_embeddings.reshape(B_Q, D)
    k = key_embeddings.reshape(N_KEYS, D)

    sim, cmax = pl.pallas_call(
        _simblock_kernel,
        grid=(N_BLK,),
        in_specs=[
            pl.BlockSpec((B_Q, D), lambda j: (0, 0)),
            pl.BlockSpec((BLK, D), lambda j: (j, 0)),
        ],
        out_specs=[
            pl.BlockSpec((1, BLK // CHUNK, B_Q, CHUNK), lambda j: (j, 0, 0, 0)),
            pl.BlockSpec((1, B_Q, BLK // CHUNK), lambda j: (j, 0, 0)),
        ],
        out_shape=[
            jax.ShapeDtypeStruct((N_BLK, BLK // CHUNK, B_Q, CHUNK), jnp.float32),
            jax.ShapeDtypeStruct((N_BLK, B_Q, BLK // CHUNK), jnp.float32),
        ],
        scratch_shapes=[pltpu.VMEM((B_Q, D), jnp.float32)],
    )(q, k)

    cidx_pad = pl.pallas_call(
        _topchunk_kernel,
        out_shape=jax.ShapeDtypeStruct((B_Q, CHUNK), jnp.int32),
    )(cmax)

    chunk_idx = cidx_pad[:, :TOPK]
    row = jnp.arange(B_Q, dtype=jnp.int32)[:, None]
    flat_cand = (chunk_idx * B_Q + row).reshape(-1)

    sim_rows = sim.reshape(B_Q * N_CHUNK, CHUNK)
    cand = _unpad_worker_rows(
        _sc_gather(sim_rows, _pad_worker_idx(flat_cand)), CHUNK)

    vout, iout = pl.pallas_call(
        _final_topk_kernel,
        out_shape=[
            jax.ShapeDtypeStruct((B_Q, CHUNK), jnp.float32),
            jax.ShapeDtypeStruct((B_Q, CHUNK), jnp.int32),
        ],
    )(cand.reshape(B_Q, TOPK * CHUNK), cidx_pad)

    top_sim = vout[:, :TOPK]
    top_idx = iout[:, :TOPK]

    retrieved = _unpad_worker_rows(
        _sc_gather(k, _pad_worker_idx(top_idx.reshape(-1))), D)
    return top_sim, top_idx, retrieved.reshape(B_Q, TOPK, 1, D)


# speedup vs baseline: 1.6268x; 1.0340x over previous
import functools

import jax
import jax.numpy as jnp
from jax import lax
from jax.experimental import pallas as pl
from jax.experimental.pallas import tpu as pltpu
from jax.experimental.pallas import tpu_sc as plsc

B_Q = 1024
D = 128
N_KEYS = 100000
BLK = 2048
N_BLK = 49
M_PAD = N_BLK * BLK
CHUNK = 128
N_CHUNK = M_PAD // CHUNK
N_CHUNK_PAD = 896
TOPK = 10
NEG = -3.0e38
BIG_I = 2**30

SC_CORES = 2
SC_SUBCORES = 16
NW = SC_CORES * SC_SUBCORES
N_IDX = B_Q * TOPK
IDX_PER_W = N_IDX // NW
IDX_ROWS = 3
IDX_PAD_W = IDX_ROWS * CHUNK


def _simblock_kernel(q_ref, k_ref, sim_ref, cmax_ref, qn_ref):
    j = pl.program_id(0)

    @pl.when(j == 0)
    def _():
        q = q_ref[...]
        n = jnp.sqrt(jnp.sum(q * q, axis=1, keepdims=True))
        qn_ref[...] = q / jnp.maximum(n, 1e-12)

    k = k_ref[...]
    kn = k / jnp.maximum(jnp.sqrt(jnp.sum(k * k, axis=1, keepdims=True)), 1e-12)
    qn = qn_ref[...]
    parts = []
    s_chunks = []
    for c in range(BLK // CHUNK):
        s_c = lax.dot_general(qn, kn[c * CHUNK:(c + 1) * CHUNK, :],
                              (((1,), (1,)), ((), ())),
                              preferred_element_type=jnp.float32)
        s_chunks.append(s_c)
        sim_ref[0, c] = s_c
        parts.append(jnp.max(s_c, axis=1, keepdims=True))
    cmax_ref[0] = jnp.concatenate(parts, axis=1)

    @pl.when(j == N_BLK - 1)
    def _():
        lane = lax.broadcasted_iota(jnp.int32, (B_Q, CHUNK), 1)
        mparts = list(parts)
        c_lo = (N_KEYS - (N_BLK - 1) * BLK) // CHUNK
        for c in range(c_lo, BLK // CHUNK):
            col = (N_BLK - 1) * BLK + c * CHUNK + lane
            s_m = jnp.where(col < N_KEYS, s_chunks[c], NEG)
            sim_ref[0, c] = s_m
            mparts[c] = jnp.max(s_m, axis=1, keepdims=True)
        cmax_ref[0] = jnp.concatenate(mparts, axis=1)


def _topchunk_kernel(cmax_ref, cidx_ref):
    parts = [cmax_ref[c] for c in range(N_BLK)]
    parts.append(jnp.full((B_Q, N_CHUNK_PAD - N_CHUNK), NEG, jnp.float32))
    v = jnp.concatenate(parts, axis=1)
    col = lax.broadcasted_iota(jnp.int32, (B_Q, N_CHUNK_PAD), 1)
    ocol = lax.broadcasted_iota(jnp.int32, (B_Q, CHUNK), 1)
    out = jnp.zeros((B_Q, CHUNK), jnp.int32)
    for t in range(TOPK):
        m = jnp.max(v, axis=1, keepdims=True)
        sel = jnp.where(v == m, col, BIG_I)
        idx = jnp.min(sel, axis=1, keepdims=True)
        out = jnp.where(ocol == t, idx, out)
        v = jnp.where(col == idx, NEG, v)
    cidx_ref[...] = out


def _final_topk_kernel(cand_ref, cidx_ref, vout_ref, iout_ref):
    cand = cand_ref[...]
    lane = lax.broadcasted_iota(jnp.int32, (B_Q, CHUNK), 1)
    keyidx = jnp.concatenate(
        [cidx_ref[:, t:t + 1] * CHUNK + lane for t in range(TOPK)], axis=1)
    ocol = lax.broadcasted_iota(jnp.int32, (B_Q, CHUNK), 1)
    vout = jnp.zeros((B_Q, CHUNK), jnp.float32)
    iout = jnp.zeros((B_Q, CHUNK), jnp.int32)
    for t in range(TOPK):
        m = jnp.max(cand, axis=1, keepdims=True)
        sel = jnp.where(cand == m, keyidx, BIG_I)
        ki = jnp.min(sel, axis=1, keepdims=True)
        vout = jnp.where(ocol == t, m, vout)
        iout = jnp.where(ocol == t, ki, iout)
        cand = jnp.where((cand == m) & (keyidx == ki), NEG, cand)
    vout_ref[...] = vout
    iout_ref[...] = iout


def _sc_gather(table, idx3d):
    d = table.shape[1]
    mesh = plsc.VectorSubcoreMesh(core_axis_name="c", subcore_axis_name="s")

    @functools.partial(
        pl.kernel, mesh=mesh,
        out_type=jax.ShapeDtypeStruct((NW * IDX_PAD_W, d), jnp.float32),
        scratch_types=[
            pltpu.VMEM((IDX_ROWS, CHUNK), jnp.int32),
            pltpu.VMEM((IDX_PAD_W, d), jnp.float32),
            pltpu.SemaphoreType.DMA,
        ],
    )
    def gather(table_hbm, idx_hbm, out_hbm, idx_v, rows_v, sem):
        wid = lax.axis_index("s") * SC_CORES + lax.axis_index("c")
        pltpu.sync_copy(idx_hbm.at[wid], idx_v)
        copies = [
            pltpu.async_copy(table_hbm.at[idx_v.at[j]],
                             rows_v.at[pl.ds(j * CHUNK, CHUNK)], sem)
            for j in range(IDX_ROWS)
        ]
        for cp in copies:
            cp.wait()
        pltpu.sync_copy(rows_v, out_hbm.at[pl.ds(wid * IDX_PAD_W, IDX_PAD_W)])

    return gather(table, idx3d)


def _pad_worker_idx(flat_idx):
    per_w = flat_idx.reshape(NW, IDX_PER_W)
    return jnp.pad(per_w, ((0, 0), (0, IDX_PAD_W - IDX_PER_W)),
                   mode="wrap").reshape(NW, IDX_ROWS, CHUNK)


def _unpad_worker_rows(rows, d):
    return rows.reshape(NW, IDX_PAD_W, d)[:, :IDX_PER_W].reshape(N_IDX, d)


def kernel(query_embeddings, key_embeddings):
    q = query_embeddings.reshape(B_Q, D)
    k = key_embeddings.reshape(N_KEYS, D)

    sim, cmax = pl.pallas_call(
        _simblock_kernel,
        grid=(N_BLK,),
        in_specs=[
            pl.BlockSpec((B_Q, D), lambda j: (0, 0)),
            pl.BlockSpec((BLK, D), lambda j: (j, 0)),
        ],
        out_specs=[
            pl.BlockSpec((1, BLK // CHUNK, B_Q, CHUNK), lambda j: (j, 0, 0, 0)),
            pl.BlockSpec((1, B_Q, BLK // CHUNK), lambda j: (j, 0, 0)),
        ],
        out_shape=[
            jax.ShapeDtypeStruct((N_BLK, BLK // CHUNK, B_Q, CHUNK), jnp.float32),
            jax.ShapeDtypeStruct((N_BLK, B_Q, BLK // CHUNK), jnp.float32),
        ],
        scratch_shapes=[pltpu.VMEM((B_Q, D), jnp.float32)],
    )(q, k)

    cidx_pad = pl.pallas_call(
        _topchunk_kernel,
        out_shape=jax.ShapeDtypeStruct((B_Q, CHUNK), jnp.int32),
    )(cmax)

    chunk_idx = cidx_pad[:, :TOPK]
    row = jnp.arange(B_Q, dtype=jnp.int32)[:, None]
    flat_cand = (chunk_idx * B_Q + row).reshape(-1)

    sim_rows = sim.reshape(B_Q * N_CHUNK, CHUNK)
    cand = _unpad_worker_rows(
        _sc_gather(sim_rows, _pad_worker_idx(flat_cand)), CHUNK)

    vout, iout = pl.pallas_call(
        _final_topk_kernel,
        out_shape=[
            jax.ShapeDtypeStruct((B_Q, CHUNK), jnp.float32),
            jax.ShapeDtypeStruct((B_Q, CHUNK), jnp.int32),
        ],
    )(cand.reshape(B_Q, TOPK * CHUNK), cidx_pad)

    top_sim = vout[:, :TOPK]
    top_idx = iout[:, :TOPK]

    retrieved = _unpad_worker_rows(
        _sc_gather(k, _pad_worker_idx(top_idx.reshape(-1))), D)
    return top_sim, top_idx, retrieved.reshape(B_Q, TOPK, 1, D)


# speedup vs baseline: 1.6670x; 1.0247x over previous
import functools

import jax
import jax.numpy as jnp
from jax import lax
from jax.experimental import pallas as pl
from jax.experimental.pallas import tpu as pltpu
from jax.experimental.pallas import tpu_sc as plsc

B_Q = 1024
D = 128
N_KEYS = 100000
BLK = 2048
N_BLK = 49
M_PAD = N_BLK * BLK
CHUNK = 128
N_CHUNK = M_PAD // CHUNK
N_CHUNK_PAD = 896
TOPK = 10
NEG = -3.0e38
BIG_I = 2**30

SC_CORES = 2
SC_SUBCORES = 16
NW = SC_CORES * SC_SUBCORES
N_IDX = B_Q * TOPK
IDX_PER_W = N_IDX // NW
def _simblock_kernel(q_ref, k_ref, sim_ref, cmax_ref, qn_ref):
    j = pl.program_id(0)

    @pl.when(j == 0)
    def _():
        q = q_ref[...]
        n = jnp.sqrt(jnp.sum(q * q, axis=1, keepdims=True))
        qn_ref[...] = q / jnp.maximum(n, 1e-12)

    k = k_ref[...]
    kn = k / jnp.maximum(jnp.sqrt(jnp.sum(k * k, axis=1, keepdims=True)), 1e-12)
    qn = qn_ref[...]
    parts = []
    s_chunks = []
    for c in range(BLK // CHUNK):
        s_c = lax.dot_general(qn, kn[c * CHUNK:(c + 1) * CHUNK, :],
                              (((1,), (1,)), ((), ())),
                              preferred_element_type=jnp.float32)
        s_chunks.append(s_c)
        sim_ref[0, c] = s_c
        parts.append(jnp.max(s_c, axis=1, keepdims=True))
    cmax_ref[0] = jnp.concatenate(parts, axis=1)

    @pl.when(j == N_BLK - 1)
    def _():
        lane = lax.broadcasted_iota(jnp.int32, (B_Q, CHUNK), 1)
        mparts = list(parts)
        c_lo = (N_KEYS - (N_BLK - 1) * BLK) // CHUNK
        for c in range(c_lo, BLK // CHUNK):
            col = (N_BLK - 1) * BLK + c * CHUNK + lane
            s_m = jnp.where(col < N_KEYS, s_chunks[c], NEG)
            sim_ref[0, c] = s_m
            mparts[c] = jnp.max(s_m, axis=1, keepdims=True)
        cmax_ref[0] = jnp.concatenate(mparts, axis=1)


def _topchunk_kernel(cmax_ref, cidx_ref):
    parts = [cmax_ref[c] for c in range(N_BLK)]
    parts.append(jnp.full((B_Q, N_CHUNK_PAD - N_CHUNK), NEG, jnp.float32))
    v = jnp.concatenate(parts, axis=1)
    col = lax.broadcasted_iota(jnp.int32, (B_Q, N_CHUNK_PAD), 1)
    ocol = lax.broadcasted_iota(jnp.int32, (B_Q, CHUNK), 1)
    out = jnp.zeros((B_Q, CHUNK), jnp.int32)
    for t in range(TOPK):
        m = jnp.max(v, axis=1, keepdims=True)
        sel = jnp.where(v == m, col, BIG_I)
        idx = jnp.min(sel, axis=1, keepdims=True)
        out = jnp.where(ocol == t, idx, out)
        v = jnp.where(col == idx, NEG, v)
    cidx_ref[...] = out


def _final_topk_kernel(cand_ref, cidx_ref, vout_ref, iout_ref):
    cand = cand_ref[...]
    lane = lax.broadcasted_iota(jnp.int32, (B_Q, CHUNK), 1)
    keyidx = jnp.concatenate(
        [cidx_ref[:, t:t + 1] * CHUNK + lane for t in range(TOPK)], axis=1)
    ocol = lax.broadcasted_iota(jnp.int32, (B_Q, CHUNK), 1)
    vout = jnp.zeros((B_Q, CHUNK), jnp.float32)
    iout = jnp.zeros((B_Q, CHUNK), jnp.int32)
    for t in range(TOPK):
        m = jnp.max(cand, axis=1, keepdims=True)
        sel = jnp.where(cand == m, keyidx, BIG_I)
        ki = jnp.min(sel, axis=1, keepdims=True)
        vout = jnp.where(ocol == t, m, vout)
        iout = jnp.where(ocol == t, ki, iout)
        cand = jnp.where((cand == m) & (keyidx == ki), NEG, cand)
    vout_ref[...] = vout
    iout_ref[...] = iout


def _sc_gather(table, idx2d):
    d = table.shape[1]
    mesh = plsc.VectorSubcoreMesh(core_axis_name="c", subcore_axis_name="s")

    @functools.partial(
        pl.kernel, mesh=mesh,
        out_type=jax.ShapeDtypeStruct((N_IDX, d), jnp.float32),
        scratch_types=[
            pltpu.VMEM((IDX_PER_W,), jnp.int32),
            pltpu.VMEM((IDX_PER_W, d), jnp.float32),
            pltpu.SemaphoreType.DMA,
        ],
    )
    def gather(table_hbm, idx_hbm, out_hbm, idx_v, rows_v, sem):
        wid = lax.axis_index("s") * SC_CORES + lax.axis_index("c")
        pltpu.sync_copy(idx_hbm.at[wid], idx_v)
        copies = [
            pltpu.async_copy(
                table_hbm.at[idx_v.at[pl.ds(off, min(CHUNK, IDX_PER_W - off))]],
                rows_v.at[pl.ds(off, min(CHUNK, IDX_PER_W - off))], sem)
            for off in range(0, IDX_PER_W, CHUNK)
        ]
        for cp in copies:
            cp.wait()
        pltpu.sync_copy(rows_v, out_hbm.at[pl.ds(wid * IDX_PER_W, IDX_PER_W)])

    return gather(table, idx2d)


def kernel(query_embeddings, key_embeddings):
    q = query_embeddings.reshape(B_Q, D)
    k = key_embeddings.reshape(N_KEYS, D)

    sim, cmax = pl.pallas_call(
        _simblock_kernel,
        grid=(N_BLK,),
        in_specs=[
            pl.BlockSpec((B_Q, D), lambda j: (0, 0)),
            pl.BlockSpec((BLK, D), lambda j: (j, 0)),
        ],
        out_specs=[
            pl.BlockSpec((1, BLK // CHUNK, B_Q, CHUNK), lambda j: (j, 0, 0, 0)),
            pl.BlockSpec((1, B_Q, BLK // CHUNK), lambda j: (j, 0, 0)),
        ],
        out_shape=[
            jax.ShapeDtypeStruct((N_BLK, BLK // CHUNK, B_Q, CHUNK), jnp.float32),
            jax.ShapeDtypeStruct((N_BLK, B_Q, BLK // CHUNK), jnp.float32),
        ],
        scratch_shapes=[pltpu.VMEM((B_Q, D), jnp.float32)],
    )(q, k)

    cidx_pad = pl.pallas_call(
        _topchunk_kernel,
        out_shape=jax.ShapeDtypeStruct((B_Q, CHUNK), jnp.int32),
    )(cmax)

    chunk_idx = cidx_pad[:, :TOPK]
    row = jnp.arange(B_Q, dtype=jnp.int32)[:, None]
    flat_cand = (chunk_idx * B_Q + row).reshape(-1)

    sim_rows = sim.reshape(B_Q * N_CHUNK, CHUNK)
    cand = _sc_gather(sim_rows, flat_cand.reshape(NW, IDX_PER_W))

    vout, iout = pl.pallas_call(
        _final_topk_kernel,
        out_shape=[
            jax.ShapeDtypeStruct((B_Q, CHUNK), jnp.float32),
            jax.ShapeDtypeStruct((B_Q, CHUNK), jnp.int32),
        ],
    )(cand.reshape(B_Q, TOPK * CHUNK), cidx_pad)

    top_sim = vout[:, :TOPK]
    top_idx = iout[:, :TOPK]

    retrieved = _sc_gather(k, top_idx.reshape(NW, IDX_PER_W))
    return top_sim, top_idx, retrieved.reshape(B_Q, TOPK, 1, D)


# speedup vs baseline: 1.8124x; 1.0872x over previous
import functools

import jax
import jax.numpy as jnp
from jax import lax
from jax.experimental import pallas as pl
from jax.experimental.pallas import tpu as pltpu
from jax.experimental.pallas import tpu_sc as plsc

B_Q = 1024
D = 128
N_KEYS = 100000
BLK = 2048
N_BLK = 49
M_PAD = N_BLK * BLK
CHUNK = 128
N_CHUNK = M_PAD // CHUNK
N_CHUNK_PAD = 896
TOPK = 10
NEG = -3.0e38
BIG_I = 2**30

SC_CORES = 2
SC_SUBCORES = 16
NW = SC_CORES * SC_SUBCORES
N_IDX = B_Q * TOPK
IDX_PER_W = N_IDX // NW
def _simblock_kernel(q_ref, k_ref, sim_ref, cmax_ref, qn_ref):
    j = pl.program_id(0)

    @pl.when(j == 0)
    def _():
        q = q_ref[...]
        n = jnp.sqrt(jnp.sum(q * q, axis=1, keepdims=True))
        qn_ref[...] = q / jnp.maximum(n, 1e-12)

    k = k_ref[...]
    kn = k / jnp.maximum(jnp.sqrt(jnp.sum(k * k, axis=1, keepdims=True)), 1e-12)
    qn = qn_ref[...]
    parts = []
    s_chunks = []
    for c in range(BLK // CHUNK):
        s_c = lax.dot_general(qn, kn[c * CHUNK:(c + 1) * CHUNK, :],
                              (((1,), (1,)), ((), ())),
                              preferred_element_type=jnp.float32)
        s_chunks.append(s_c)
        sim_ref[0, c] = s_c
        parts.append(jnp.max(s_c, axis=1, keepdims=True))
    cmax_ref[0] = jnp.transpose(jnp.concatenate(parts, axis=1), (1, 0))

    @pl.when(j == N_BLK - 1)
    def _():
        lane = lax.broadcasted_iota(jnp.int32, (B_Q, CHUNK), 1)
        mparts = list(parts)
        c_lo = (N_KEYS - (N_BLK - 1) * BLK) // CHUNK
        for c in range(c_lo, BLK // CHUNK):
            col = (N_BLK - 1) * BLK + c * CHUNK + lane
            s_m = jnp.where(col < N_KEYS, s_chunks[c], NEG)
            sim_ref[0, c] = s_m
            mparts[c] = jnp.max(s_m, axis=1, keepdims=True)
        cmax_ref[0] = jnp.transpose(jnp.concatenate(mparts, axis=1), (1, 0))


def _topchunk_kernel(cmax_ref, cidx_ref):
    v = jnp.concatenate([cmax_ref[c] for c in range(N_BLK)], axis=0)
    rowc = lax.broadcasted_iota(jnp.int32, (N_CHUNK, B_Q), 0)
    orow = lax.broadcasted_iota(jnp.int32, (24, B_Q), 0)
    out = jnp.zeros((24, B_Q), jnp.int32)
    for t in range(TOPK):
        m = jnp.max(v, axis=0, keepdims=True)
        sel = jnp.where(v == m, rowc, BIG_I)
        idx = jnp.min(sel, axis=0, keepdims=True)
        out = jnp.where(orow == t, idx, out)
        v = jnp.where(rowc == idx, NEG, v)
    cidx_ref[...] = out


def _final_topk_kernel(cand_ref, cidx_ref, vout_ref, iout_ref):
    cand = cand_ref[...]
    lane = lax.broadcasted_iota(jnp.int32, (B_Q, CHUNK), 1)
    keyidx = jnp.concatenate(
        [cidx_ref[:, t:t + 1] * CHUNK + lane for t in range(TOPK)], axis=1)
    ocol = lax.broadcasted_iota(jnp.int32, (B_Q, CHUNK), 1)
    vout = jnp.zeros((B_Q, CHUNK), jnp.float32)
    iout = jnp.zeros((B_Q, CHUNK), jnp.int32)
    for t in range(TOPK):
        m = jnp.max(cand, axis=1, keepdims=True)
        sel = jnp.where(cand == m, keyidx, BIG_I)
        ki = jnp.min(sel, axis=1, keepdims=True)
        vout = jnp.where(ocol == t, m, vout)
        iout = jnp.where(ocol == t, ki, iout)
        cand = jnp.where((cand == m) & (keyidx == ki), NEG, cand)
    vout_ref[...] = vout
    iout_ref[...] = iout


def _sc_gather(table, idx2d):
    d = table.shape[1]
    mesh = plsc.VectorSubcoreMesh(core_axis_name="c", subcore_axis_name="s")

    @functools.partial(
        pl.kernel, mesh=mesh,
        out_type=jax.ShapeDtypeStruct((N_IDX, d), jnp.float32),
        scratch_types=[
            pltpu.VMEM((IDX_PER_W,), jnp.int32),
            pltpu.VMEM((IDX_PER_W, d), jnp.float32),
            pltpu.SemaphoreType.DMA,
        ],
    )
    def gather(table_hbm, idx_hbm, out_hbm, idx_v, rows_v, sem):
        wid = lax.axis_index("s") * SC_CORES + lax.axis_index("c")
        pltpu.sync_copy(idx_hbm.at[wid], idx_v)
        copies = [
            pltpu.async_copy(
                table_hbm.at[idx_v.at[pl.ds(off, min(CHUNK, IDX_PER_W - off))]],
                rows_v.at[pl.ds(off, min(CHUNK, IDX_PER_W - off))], sem)
            for off in range(0, IDX_PER_W, CHUNK)
        ]
        for cp in copies:
            cp.wait()
        pltpu.sync_copy(rows_v, out_hbm.at[pl.ds(wid * IDX_PER_W, IDX_PER_W)])

    return gather(table, idx2d)


def kernel(query_embeddings, key_embeddings):
    q = query_embeddings.reshape(B_Q, D)
    k = key_embeddings.reshape(N_KEYS, D)

    sim, cmax = pl.pallas_call(
        _simblock_kernel,
        grid=(N_BLK,),
        in_specs=[
            pl.BlockSpec((B_Q, D), lambda j: (0, 0)),
            pl.BlockSpec((BLK, D), lambda j: (j, 0)),
        ],
        out_specs=[
            pl.BlockSpec((1, BLK // CHUNK, B_Q, CHUNK), lambda j: (j, 0, 0, 0)),
            pl.BlockSpec((1, BLK // CHUNK, B_Q), lambda j: (j, 0, 0)),
        ],
        out_shape=[
            jax.ShapeDtypeStruct((N_BLK, BLK // CHUNK, B_Q, CHUNK), jnp.float32),
            jax.ShapeDtypeStruct((N_BLK, BLK // CHUNK, B_Q), jnp.float32),
        ],
        scratch_shapes=[pltpu.VMEM((B_Q, D), jnp.float32)],
    )(q, k)

    cidx_t = pl.pallas_call(
        _topchunk_kernel,
        out_shape=jax.ShapeDtypeStruct((24, B_Q), jnp.int32),
    )(cmax)

    chunk_idx = cidx_t[:TOPK].transpose(1, 0)
    cidx_pad = jnp.pad(chunk_idx, ((0, 0), (0, 6)))
    row = jnp.arange(B_Q, dtype=jnp.int32)[:, None]
    flat_cand = (chunk_idx * B_Q + row).reshape(-1)

    sim_rows = sim.reshape(B_Q * N_CHUNK, CHUNK)
    cand = _sc_gather(sim_rows, flat_cand.reshape(NW, IDX_PER_W))

    vout, iout = pl.pallas_call(
        _final_topk_kernel,
        out_shape=[
            jax.ShapeDtypeStruct((B_Q, CHUNK), jnp.float32),
            jax.ShapeDtypeStruct((B_Q, CHUNK), jnp.int32),
        ],
    )(cand.reshape(B_Q, TOPK * CHUNK), cidx_pad)

    top_sim = vout[:, :TOPK]
    top_idx = iout[:, :TOPK]

    retrieved = _sc_gather(k, top_idx.reshape(NW, IDX_PER_W))
    return top_sim, top_idx, retrieved.reshape(B_Q, TOPK, 1, D)
